# Initial kernel scaffold; baseline (speedup 1.0000x reference)
import jax
import jax.numpy as jnp
from jax.experimental import pallas as pl


def kernel(x, edge_index, W1, b1, W2, b2, Wc1, bc1, Wc2, bc2):
    raise NotImplementedError("write your pallas kernel here")


# speedup vs baseline: 10.7009x; 10.7009x over previous
import functools

import jax
import jax.numpy as jnp
from jax import lax
from jax.experimental import pallas as pl
from jax.experimental.pallas import tpu as pltpu
from jax.experimental.pallas import tpu_sc as plsc

N = 10000
E = 320000
D = 128

NC = 2
NS = 16
NW = NC * NS
CH = 80
EPW = E // NW
CPW = EPW // CH
NPAD = 10240
RPT = NPAD // NS
ZR = 128

BLK = 400
GRID = N // BLK

_MESH = plsc.VectorSubcoreMesh(core_axis_name="c", subcore_axis_name="s")
_HIGH = jax.lax.Precision.HIGHEST
_SC_PARAMS = pltpu.CompilerParams(needs_layout_passes=False)


@functools.partial(
    pl.kernel,
    out_type=jax.ShapeDtypeStruct((2, NW, 1, N), jnp.float32),
    mesh=_MESH,
    scratch_types=[
        pltpu.VMEM((CPW, CH), jnp.int32),
        pltpu.VMEM((N,), jnp.float32),
    ],
    compiler_params=_SC_PARAMS,
)
def _deg_kernel(src_hbm, dst_hbm, out_hbm, idx_v, hist_v):
    c = lax.axis_index("c")
    s = lax.axis_index("s")
    wid = s * NC + c
    ones = jnp.full((16,), 1.0, dtype=jnp.float32)
    zeros = jnp.zeros((16,), dtype=jnp.float32)

    def one_direction(edge_hbm, out_row):
        pltpu.sync_copy(edge_hbm.at[wid], idx_v)

        def zbody(i, _):
            hist_v[pl.ds(i * 16, 16)] = zeros
            return _

        lax.fori_loop(0, N // 16, zbody, None)

        def hbody(r, _):
            for k in range(CH // 16):
                v = idx_v[r, pl.ds(k * 16, 16)]
                plsc.addupdate_scatter(hist_v, [v], ones)
            return _

        lax.fori_loop(0, CPW, hbody, None)
        pltpu.sync_copy(hist_v, out_row)

    one_direction(src_hbm, out_hbm.at[0, wid, 0])
    one_direction(dst_hbm, out_hbm.at[1, wid, 0])


@functools.partial(
    pl.kernel,
    out_type=jax.ShapeDtypeStruct((NC, NPAD, D), jnp.float32),
    mesh=_MESH,
    scratch_types=[
        pltpu.VMEM((CPW, CH), jnp.int32),
        pltpu.VMEM((CPW, CH), jnp.int32),
        pltpu.VMEM((CH, D), jnp.float32),
        pltpu.VMEM_SHARED((NPAD, D), jnp.float32),
        pltpu.SemaphoreType.DMA,
    ],
    compiler_params=_SC_PARAMS,
)
def _edge_kernel(hs_hbm, src_hbm, dst_hbm, out_hbm, isrc, idst, rows,
                 agg, sem):
    c = lax.axis_index("c")
    s = lax.axis_index("s")
    wid = s * NC + c
    zeros = jnp.zeros((16,), dtype=jnp.float32)

    def zbody(r, _):
        for k in range(D // 16):
            rows[r, pl.ds(k * 16, 16)] = zeros
        return _

    lax.fori_loop(0, CH, zbody, None)
    for j in range(RPT // CH):
        pltpu.sync_copy(rows, agg.at[pl.ds(s * RPT + j * CH, CH)])
    plsc.subcore_barrier()

    pltpu.sync_copy(src_hbm.at[wid], isrc)
    pltpu.sync_copy(dst_hbm.at[wid], idst)

    def body(g, _):
        pltpu.async_copy(hs_hbm.at[isrc.at[g]], rows, sem).wait()
        pltpu.sync_copy(rows, agg.at[idst.at[g]], add=True)
        return _

    lax.fori_loop(0, CPW, body, None)
    plsc.subcore_barrier()

    pltpu.sync_copy(agg.at[pl.ds(s * RPT, RPT)], out_hbm.at[c, pl.ds(s * RPT, RPT)])


def _norms_body(deg_ref, out_ref):
    d = jnp.sum(deg_ref[...], axis=1)
    out_ref[...] = jax.lax.rsqrt(jnp.maximum(d, 1.0))


def _norms_tc(deg):
    return pl.pallas_call(
        _norms_body,
        out_shape=jax.ShapeDtypeStruct((2, N), jnp.float32),
    )(deg)


def _mm_scale_body(x_ref, w_ref, ns_ref, out_ref):
    h = jnp.dot(x_ref[...], w_ref[...], precision=_HIGH,
                preferred_element_type=jnp.float32)
    out_ref[...] = h * ns_ref[...]


def _mm_scale_tc(x, w, ns_col):
    return pl.pallas_call(
        _mm_scale_body,
        grid=(GRID,),
        in_specs=[
            pl.BlockSpec((BLK, D), lambda i: (i, 0)),
            pl.BlockSpec((D, D), lambda i: (0, 0)),
            pl.BlockSpec((BLK, 1), lambda i: (i, 0)),
        ],
        out_specs=pl.BlockSpec((BLK, D), lambda i: (i, 0)),
        out_shape=jax.ShapeDtypeStruct((N, D), jnp.float32),
    )(x, w, ns_col)


def _mid_body(p_ref, nd_ref, b_ref, w_ref, ns_ref, out_ref):
    h = jax.nn.relu((p_ref[0] + p_ref[1]) * nd_ref[...] + b_ref[...])
    h2 = jnp.dot(h, w_ref[...], precision=_HIGH,
                 preferred_element_type=jnp.float32)
    out_ref[...] = h2 * ns_ref[...]


def _mid_tc(p, nd_col, b_row, w, ns_col):
    return pl.pallas_call(
        _mid_body,
        grid=(GRID,),
        in_specs=[
            pl.BlockSpec((NC, BLK, D), lambda i: (0, i, 0)),
            pl.BlockSpec((BLK, 1), lambda i: (i, 0)),
            pl.BlockSpec((1, D), lambda i: (0, 0)),
            pl.BlockSpec((D, D), lambda i: (0, 0)),
            pl.BlockSpec((BLK, 1), lambda i: (i, 0)),
        ],
        out_specs=pl.BlockSpec((BLK, D), lambda i: (i, 0)),
        out_shape=jax.ShapeDtypeStruct((N, D), jnp.float32),
    )(p, nd_col, b_row, w, ns_col)


def _final_body(p_ref, nd_ref, b_ref, wc1_ref, bc1_ref, wc2_ref, bc2_ref,
                ne_ref, ge_ref, lg_ref, acc_ref):
    i = pl.program_id(0)
    ne = jax.nn.relu((p_ref[0] + p_ref[1]) * nd_ref[...] + b_ref[...])
    ne_ref[...] = ne

    @pl.when(i == 0)
    def _():
        acc_ref[...] = jnp.zeros_like(acc_ref)

    acc_ref[...] += jnp.sum(ne, axis=0, keepdims=True)

    @pl.when(i == GRID - 1)
    def _():
        ge = acc_ref[...] * (1.0 / N)
        ge_ref[...] = ge
        hc = jax.nn.relu(jnp.dot(ge, wc1_ref[...], precision=_HIGH,
                                 preferred_element_type=jnp.float32)
                         + bc1_ref[...])
        lg_ref[...] = jnp.dot(hc, wc2_ref[...], precision=_HIGH,
                              preferred_element_type=jnp.float32) + bc2_ref[...]


def _final_tc(p, nd_col, b_row, wc1, bc1_row, wc2, bc2_row):
    return pl.pallas_call(
        _final_body,
        grid=(GRID,),
        in_specs=[
            pl.BlockSpec((NC, BLK, D), lambda i: (0, i, 0)),
            pl.BlockSpec((BLK, 1), lambda i: (i, 0)),
            pl.BlockSpec((1, D), lambda i: (0, 0)),
            pl.BlockSpec((D, D), lambda i: (0, 0)),
            pl.BlockSpec((1, D), lambda i: (0, 0)),
            pl.BlockSpec((D, 1), lambda i: (0, 0)),
            pl.BlockSpec((1, 1), lambda i: (0, 0)),
        ],
        out_specs=[
            pl.BlockSpec((BLK, D), lambda i: (i, 0)),
            pl.BlockSpec((1, D), lambda i: (0, 0)),
            pl.BlockSpec((1, 1), lambda i: (0, 0)),
        ],
        out_shape=[
            jax.ShapeDtypeStruct((N, D), jnp.float32),
            jax.ShapeDtypeStruct((1, D), jnp.float32),
            jax.ShapeDtypeStruct((1, 1), jnp.float32),
        ],
        scratch_shapes=[pltpu.VMEM((1, D), jnp.float32)],
    )(p, nd_col, b_row, wc1, bc1_row, wc2, bc2_row)


def kernel(x, edge_index, W1, b1, W2, b2, Wc1, bc1, Wc2, bc2):
    src2 = edge_index[0].reshape(NW, CPW, CH)
    dst2 = edge_index[1].reshape(NW, CPW, CH)

    deg = _deg_kernel(src2, dst2).reshape(2, NW, N)
    norms = _norms_tc(deg)
    ns_col = norms[0].reshape(N, 1)
    nd_col = norms[1].reshape(N, 1)
    b1r = b1.reshape(1, D)
    b2r = b2.reshape(1, D)
    bc1r = bc1.reshape(1, D)
    bc2r = bc2.reshape(1, 1)

    hs1 = _mm_scale_tc(x, W1, ns_col)
    p1 = _edge_kernel(hs1, src2, dst2)
    hs2 = _mid_tc(p1, nd_col, b1r, W2, ns_col)
    p2 = _edge_kernel(hs2, src2, dst2)
    node_emb, graph_emb, logits = _final_tc(p2, nd_col, b2r, Wc1, bc1r,
                                            Wc2, bc2r)
    return (node_emb, graph_emb, logits)
